# baseline (speedup 1.0000x reference)
import functools

import jax
import jax.numpy as jnp
from jax import lax
from jax.experimental import pallas as pl
from jax.experimental.pallas import tpu as pltpu
from jax.experimental.pallas import tpu_sc as plsc

ROW = 272
ROW_W = 136
ROW_WPAD = 144
BATCH = 16384
NUM_CORES = 2
NUM_SUBCORES = 16
NW = NUM_CORES * NUM_SUBCORES
B_PER_W = BATCH // NW
SEC = 64
NSEC = B_PER_W // SEC
NBUF = 4

_mesh = plsc.VectorSubcoreMesh(core_axis_name="c", subcore_axis_name="s")


@functools.partial(
    pl.kernel,
    mesh=_mesh,
    out_type=jax.ShapeDtypeStruct((BATCH, ROW_WPAD), jnp.uint32),
    scratch_types=[
        pltpu.VMEM((B_PER_W,), jnp.int32),
        pltpu.VMEM((NBUF, SEC, ROW_WPAD), jnp.uint32),
        pltpu.SemaphoreType.DMA,
        pltpu.SemaphoreType.DMA,
    ],
)
def _gather_kernel(idx_hbm, table_hbm, out_hbm, idx_v, rows_v, gsem, wsem):
    wid = lax.axis_index("s") * NUM_CORES + lax.axis_index("c")
    base = wid * B_PER_W
    pltpu.sync_copy(idx_hbm.at[wid], idx_v)

    writes = [None] * NSEC
    for s in range(NSEC):
        if s >= NBUF:
            writes[s - NBUF].wait()
        b = s % NBUF
        fetches = []
        for i in range(SEC):
            if i % 16 == 0:
                v = idx_v[pl.ds(s * SEC + i, 16)]
            r = v[i % 16]
            fetches.append(pltpu.async_copy(
                table_hbm.at[pl.ds(r, 1)], rows_v.at[b].at[pl.ds(i, 1)],
                gsem))
        for f in fetches:
            f.wait()
        writes[s] = pltpu.async_copy(
            rows_v.at[b], out_hbm.at[pl.ds(base + s * SEC, SEC)], wsem)
    for s in range(NSEC - NBUF, NSEC):
        writes[s].wait()


def _pack_bf16_pairs(w):
    u = lax.bitcast_convert_type(w, jnp.uint32)
    bf = (u + jnp.uint32(0x8000)) >> 16
    lo = bf[:, 0::2]
    hi = bf[:, 1::2]
    packed = lo | (hi << 16)
    return jnp.pad(packed, ((0, 0), (0, ROW_WPAD - ROW_W)))


def kernel(x_cate, cate_embedding_weight):
    idx = x_cate.astype(jnp.int32).reshape(NW, B_PER_W)
    packed = _pack_bf16_pairs(cate_embedding_weight)
    out = _gather_kernel(idx, packed)[:, :ROW_W]
    lo = lax.bitcast_convert_type(out << 16, jnp.float32)
    hi = lax.bitcast_convert_type(out & jnp.uint32(0xFFFF0000), jnp.float32)
    return jnp.stack([lo, hi], axis=-1).reshape(BATCH, ROW)


# speedup vs baseline: 35.6245x; 35.6245x over previous
import functools

import jax
import jax.numpy as jnp
from jax import lax
from jax.experimental import pallas as pl
from jax.experimental.pallas import tpu as pltpu
from jax.experimental.pallas import tpu_sc as plsc

ROW = 272
BATCH = 16384
NUM_CORES = 2
NUM_SUBCORES = 16
NW = NUM_CORES * NUM_SUBCORES
B_PER_W = BATCH // NW
SEC = 64
NSEC = B_PER_W // SEC
NBUF = 4

_mesh = plsc.VectorSubcoreMesh(core_axis_name="c", subcore_axis_name="s")


@functools.partial(
    pl.kernel,
    mesh=_mesh,
    out_type=jax.ShapeDtypeStruct((BATCH, ROW), jnp.float32),
    scratch_types=[
        pltpu.VMEM((B_PER_W,), jnp.int32),
        pltpu.VMEM((NBUF, SEC, ROW), jnp.float32),
        pltpu.SemaphoreType.DMA,
        pltpu.SemaphoreType.DMA,
    ],
)
def _gather_kernel(idx_hbm, table_hbm, out_hbm, idx_v, rows_v, gsem, wsem):
    wid = lax.axis_index("s") * NUM_CORES + lax.axis_index("c")
    base = wid * B_PER_W
    pltpu.sync_copy(idx_hbm.at[wid], idx_v)

    writes = [None] * NSEC
    for s in range(NSEC):
        if s >= NBUF:
            writes[s - NBUF].wait()
        b = s % NBUF
        fetches = []
        for i in range(SEC):
            if i % 16 == 0:
                v = idx_v[pl.ds(s * SEC + i, 16)]
            r = v[i % 16]
            fetches.append(pltpu.async_copy(
                table_hbm.at[pl.ds(r, 1)], rows_v.at[b].at[pl.ds(i, 1)],
                gsem))
        for f in fetches:
            f.wait()
        writes[s] = pltpu.async_copy(
            rows_v.at[b], out_hbm.at[pl.ds(base + s * SEC, SEC)], wsem)
    for s in range(NSEC - NBUF, NSEC):
        writes[s].wait()


def kernel(x_cate, cate_embedding_weight):
    idx = x_cate.astype(jnp.int32).reshape(NW, B_PER_W)
    return _gather_kernel(idx, cate_embedding_weight)
